# baseline (speedup 1.0000x reference)
import jax
import jax.numpy as jnp
from jax.experimental import pallas as pl

N_NODES = 10000


def _edge_conv(x, edge_index, W1, b1, W2, b2):
    src = edge_index[0]
    dst = edge_index[1]
    x_i = x[dst]
    x_j = x[src]
    m = jnp.concatenate([x_i, x_j - x_i], axis=-1)
    hp = jax.lax.Precision.HIGHEST
    def bfdot(a, b):
        return jnp.dot(a.astype(jnp.bfloat16), b.astype(jnp.bfloat16),
                       precision=hp, preferred_element_type=jnp.float32)
    h = bfdot(jax.nn.relu(bfdot(m, W1) + b1), W2) + b2
    out = jax.ops.segment_max(h, dst, num_segments=N_NODES)
    return jnp.where(jnp.isneginf(out), 0.0, out)


def kernel(x, edge_index, edge_attr, We1, be1, We2, be2, Wc1, bc1, Wc2, bc2,
           Wd1, bd1, Wd2, bd2):
    h = _edge_conv(x, edge_index, We1, be1, We2, be2)
    for _ in range(8):
        h = _edge_conv(h, edge_index, Wc1, bc1, Wc2, bc2)
    out = _edge_conv(h, edge_index, Wd1, bd1, Wd2, bd2)
    return (out, edge_attr)


# speedup vs baseline: 1.7748x; 1.7748x over previous
import functools

import jax
import jax.numpy as jnp
from jax import lax
from jax.experimental import pallas as pl
from jax.experimental.pallas import tpu as pltpu
from jax.experimental.pallas import tpu_sc as plsc

N_NODES = 10000
N_EDGES = 320000
HID = 64
NT = 32
RPT = 320
NP = NT * RPT
EPW = N_EDGES // NT
GC = 400
GSUB = 80
SC2 = 512
EPAD = N_EDGES + 2 * SC2

_NEG_INF = float("-inf")


def _bfdot(a, b):
    return jnp.dot(a.astype(jnp.bfloat16), b.astype(jnp.bfloat16),
                   preferred_element_type=jnp.float32)


def _prep_body(h_ref, hs2_ref, t3_ref):
    h = h_ref[:, :HID]
    z = jnp.zeros_like(h)
    hs2_ref[...] = jnp.concatenate([z, h], axis=1)
    t3_ref[...] = jnp.concatenate([h, -h], axis=1)


def _prep(h128):
    n = h128.shape[0]
    bm = 1024
    return pl.pallas_call(
        _prep_body,
        grid=(n // bm,),
        in_specs=[pl.BlockSpec((bm, 128), lambda i: (i, 0))],
        out_specs=[
            pl.BlockSpec((bm, 128), lambda i: (i, 0)),
            pl.BlockSpec((bm, 128), lambda i: (i, 0)),
        ],
        out_shape=[
            jax.ShapeDtypeStruct((n, 128), jnp.float32),
            jax.ShapeDtypeStruct((n, 128), jnp.float32),
        ],
    )(h128)


def _mlp_body(m_ref, w1_ref, b1_ref, w2_ref, b2_ref, h_ref):
    p1 = _bfdot(m_ref[...], w1_ref[...]) + b1_ref[...]
    h_ref[...] = _bfdot(jax.nn.relu(p1), w2_ref[...]) + b2_ref[...]


def _edge_mlp(M, W1, b1, W2, b2):
    e, w = M.shape
    bm = 512
    return pl.pallas_call(
        _mlp_body,
        grid=(e // bm,),
        in_specs=[
            pl.BlockSpec((bm, w), lambda i: (i, 0)),
            pl.BlockSpec((w, HID), lambda i: (0, 0)),
            pl.BlockSpec((1, HID), lambda i: (0, 0)),
            pl.BlockSpec((HID, HID), lambda i: (0, 0)),
            pl.BlockSpec((1, HID), lambda i: (0, 0)),
        ],
        out_specs=pl.BlockSpec((bm, HID), lambda i: (i, 0)),
        out_shape=jax.ShapeDtypeStruct((EPAD, HID), jnp.float32),
    )(M, W1, b1[None, :], W2, b2[None, :])


def _mlp1_body(ml_ref, mr_ref, w1_ref, b1_ref, w2_ref, b2_ref, h_ref):
    m = jnp.concatenate([ml_ref[...], mr_ref[...]], axis=1)
    p1 = _bfdot(m, w1_ref[...]) + b1_ref[...]
    h_ref[...] = _bfdot(jax.nn.relu(p1), w2_ref[...]) + b2_ref[...]


def _edge_mlp1(ML, MR, W1, b1, W2, b2):
    e = ML.shape[0]
    bm = 512
    return pl.pallas_call(
        _mlp1_body,
        grid=(e // bm,),
        in_specs=[
            pl.BlockSpec((bm, 128), lambda i: (i, 0)),
            pl.BlockSpec((bm, 128), lambda i: (i, 0)),
            pl.BlockSpec((256, HID), lambda i: (0, 0)),
            pl.BlockSpec((1, HID), lambda i: (0, 0)),
            pl.BlockSpec((HID, HID), lambda i: (0, 0)),
            pl.BlockSpec((1, HID), lambda i: (0, 0)),
        ],
        out_specs=pl.BlockSpec((bm, HID), lambda i: (i, 0)),
        out_shape=jax.ShapeDtypeStruct((EPAD, HID), jnp.float32),
    )(ML, MR, W1, b1[None, :], W2, b2[None, :])


def _wid():
    return lax.axis_index("s") * 2 + lax.axis_index("c")


def _gather_body(hs2_hbm, t3_hbm, src_hbm, dst_hbm, m_hbm,
                 idxs_v, idxd_v, buf_v, sem1):
    wid = _wid()
    ebase = wid * EPW

    def chunk(i, _):
        base = ebase + i * GC
        pltpu.sync_copy(src_hbm.at[pl.ds(base, GC)], idxs_v)
        pltpu.sync_copy(dst_hbm.at[pl.ds(base, GC)], idxd_v)
        handles = []
        for j in range(GC // GSUB):
            s = pl.ds(j * GSUB, GSUB)
            handles.append(
                pltpu.async_copy(hs2_hbm.at[idxs_v.at[s]], buf_v.at[s], sem1))
        for d in handles:
            d.wait()
        handles = []
        for j in range(GC // GSUB):
            s = pl.ds(j * GSUB, GSUB)
            handles.append(
                pltpu.async_copy(t3_hbm.at[idxd_v.at[s]], buf_v.at[s], sem1,
                                 add=True))
        for d in handles:
            d.wait()
        pltpu.sync_copy(buf_v, m_hbm.at[pl.ds(base, GC)])
        return ()

    lax.fori_loop(0, EPW // GC, chunk, (), unroll=False)


def _gather(HS2, T3, src_s, dst_s):
    mesh = plsc.VectorSubcoreMesh(core_axis_name="c", subcore_axis_name="s")
    kern = pl.kernel(
        _gather_body,
        out_type=jax.ShapeDtypeStruct((N_EDGES, 128), jnp.float32),
        mesh=mesh,
        scratch_types=[
            pltpu.VMEM((GC,), jnp.int32),
            pltpu.VMEM((GC,), jnp.int32),
            pltpu.VMEM((GC, 128), jnp.float32),
            pltpu.SemaphoreType.DMA,
        ],
    )
    return kern(HS2, T3, src_s, dst_s)


def _gather1_body(x_hbm, negx_hbm, src_hbm, dst_hbm, ml_hbm, mr_hbm,
                  idxs_v, idxd_v, bufl_v, bufr_v, sem1, sem2):
    wid = _wid()
    ebase = wid * EPW

    def chunk(i, _):
        base = ebase + i * GC
        pltpu.sync_copy(src_hbm.at[pl.ds(base, GC)], idxs_v)
        pltpu.sync_copy(dst_hbm.at[pl.ds(base, GC)], idxd_v)
        handles = []
        for j in range(GC // GSUB):
            s = pl.ds(j * GSUB, GSUB)
            handles.append(
                pltpu.async_copy(x_hbm.at[idxd_v.at[s]], bufl_v.at[s], sem1))
            handles.append(
                pltpu.async_copy(x_hbm.at[idxs_v.at[s]], bufr_v.at[s], sem2))
        for d in handles:
            d.wait()
        handles = []
        for j in range(GC // GSUB):
            s = pl.ds(j * GSUB, GSUB)
            handles.append(
                pltpu.async_copy(negx_hbm.at[idxd_v.at[s]], bufr_v.at[s], sem2,
                                 add=True))
        for d in handles:
            d.wait()
        pltpu.sync_copy(bufl_v, ml_hbm.at[pl.ds(base, GC)])
        pltpu.sync_copy(bufr_v, mr_hbm.at[pl.ds(base, GC)])
        return ()

    lax.fori_loop(0, EPW // GC, chunk, (), unroll=False)


def _gather1(xp, negx, src_s, dst_s):
    mesh = plsc.VectorSubcoreMesh(core_axis_name="c", subcore_axis_name="s")
    kern = pl.kernel(
        _gather1_body,
        out_type=[
            jax.ShapeDtypeStruct((N_EDGES, 128), jnp.float32),
            jax.ShapeDtypeStruct((N_EDGES, 128), jnp.float32),
        ],
        mesh=mesh,
        scratch_types=[
            pltpu.VMEM((GC,), jnp.int32),
            pltpu.VMEM((GC,), jnp.int32),
            pltpu.VMEM((GC, 128), jnp.float32),
            pltpu.VMEM((GC, 128), jnp.float32),
            pltpu.SemaphoreType.DMA,
            pltpu.SemaphoreType.DMA,
        ],
    )
    return kern(xp, negx, src_s, dst_s)


def _scatter_body(hm_hbm, dst_hbm, starts_hbm, out_hbm,
                  startv, idxb, hbuf, acc, sem):
    wid = _wid()
    pltpu.sync_copy(starts_hbm, startv)
    sv = startv[pl.ds(wid * 8, 16)]
    s0 = sv[0]
    s1 = sv[1]

    neg = jnp.full((16,), _NEG_INF, dtype=jnp.float32)

    def initrow(i, _):
        for c in range(8):
            acc[i, pl.ds(c * 16, 16)] = neg
        return ()

    lax.fori_loop(0, RPT + 1, initrow, (), unroll=False)

    eb0 = (s0 // 8) * 8
    nch = (s1 - eb0 + SC2 - 1) // SC2
    nbase = wid * RPT
    gi = lax.broadcasted_iota(jnp.int32, (16,), 0)

    def chunk(k, _):
        eb = eb0 + k * SC2
        pltpu.sync_copy(dst_hbm.at[pl.ds(eb, SC2)], idxb)
        pltpu.sync_copy(hm_hbm.at[pl.ds(eb, SC2)], hbuf)

        def group(g, _):
            idxv = idxb[pl.ds(g * 16, 16)] - nbase
            inb = (idxv >= 0) & (idxv < RPT) & ((eb + g * 16 + gi) < s1)
            rowv = jnp.where(inb, idxv, RPT)
            for l in range(16):
                row = rowv[l]
                for c in range(4):
                    hv = hbuf[g * 16 + l, pl.ds(c * 16, 16)]
                    av = acc[row, pl.ds(c * 16, 16)]
                    acc[row, pl.ds(c * 16, 16)] = jnp.maximum(av, hv)
            return ()

        lax.fori_loop(0, SC2 // 16, group, (), unroll=False)
        return ()

    lax.fori_loop(0, nch, chunk, (), unroll=False)

    zero = jnp.zeros((16,), jnp.float32)

    def fixrow(i, _):
        for c in range(8):
            v = acc[i, pl.ds(c * 16, 16)]
            acc[i, pl.ds(c * 16, 16)] = jnp.where(v == _NEG_INF, zero, v)
        return ()

    lax.fori_loop(0, RPT, fixrow, (), unroll=False)
    pltpu.sync_copy(acc.at[pl.ds(0, RPT)], out_hbm.at[pl.ds(wid * RPT, RPT)])


def _scatter_max(Hm, dst_s_pad, starts):
    mesh = plsc.VectorSubcoreMesh(core_axis_name="c", subcore_axis_name="s")
    kern = pl.kernel(
        _scatter_body,
        out_type=jax.ShapeDtypeStruct((NP, 128), jnp.float32),
        mesh=mesh,
        scratch_types=[
            pltpu.VMEM((272,), jnp.int32),
            pltpu.VMEM((SC2,), jnp.int32),
            pltpu.VMEM((SC2, HID), jnp.float32),
            pltpu.VMEM((RPT + 1, 128), jnp.float32),
            pltpu.SemaphoreType.DMA,
        ],
    )
    return kern(Hm, dst_s_pad, starts)


def _layer(h128, src_s, dst_s, dst_s_pad, starts, W1, b1, W2, b2):
    HS2, T3 = _prep(h128)
    M = _gather(HS2, T3, src_s, dst_s)
    Hm = _edge_mlp(M, W1, b1, W2, b2)
    return _scatter_max(Hm, dst_s_pad, starts)


def kernel(x, edge_index, edge_attr, We1, be1, We2, be2, Wc1, bc1, Wc2, bc2,
           Wd1, bd1, Wd2, bd2):
    src = edge_index[0].astype(jnp.int32)
    dst = edge_index[1].astype(jnp.int32)

    dst_s, order = lax.sort_key_val(dst, lax.iota(jnp.int32, N_EDGES))
    src_s = jnp.take(src, order)
    dst_s_pad = jnp.full((EPAD,), NP, jnp.int32).at[:N_EDGES].set(dst_s)
    bnds = jnp.searchsorted(dst_s, RPT * jnp.arange(33, dtype=jnp.int32),
                            side="left").astype(jnp.int32)
    t8 = 8 * jnp.arange(32, dtype=jnp.int32)
    starts = (jnp.zeros((272,), jnp.int32)
              .at[t8].set(bnds[:32])
              .at[t8 + 1].set(bnds[1:33]))

    xp = jnp.zeros((NP, 128), jnp.float32).at[:N_NODES].set(x)
    ML, MR = _gather1(xp, -xp, src_s, dst_s)
    Hm1 = _edge_mlp1(ML, MR, We1, be1, We2, be2)
    h = _scatter_max(Hm1, dst_s_pad, starts)

    for _ in range(8):
        h = _layer(h, src_s, dst_s, dst_s_pad, starts, Wc1, bc1, Wc2, bc2)

    Wd2p = jnp.zeros((HID, HID), jnp.float32).at[:, 0].set(Wd2[:, 0])
    bd2p = jnp.zeros((HID,), jnp.float32).at[0].set(bd2[0])
    h = _layer(h, src_s, dst_s, dst_s_pad, starts, Wd1, bd1, Wd2p, bd2p)

    out = h[:N_NODES, :1]
    return (out, edge_attr)


# speedup vs baseline: 1.8755x; 1.0568x over previous
import functools

import jax
import jax.numpy as jnp
from jax import lax
from jax.experimental import pallas as pl
from jax.experimental.pallas import tpu as pltpu
from jax.experimental.pallas import tpu_sc as plsc

N_NODES = 10000
N_EDGES = 320000
HID = 64
NT = 32
RPT = 320
NP = NT * RPT
EPW = N_EDGES // NT
GC = 400
GSUB = 80
SC2 = 512
EPAD = N_EDGES + 2 * SC2

_NEG_INF = float("-inf")


def _bfdot(a, b):
    return jnp.dot(a.astype(jnp.bfloat16), b.astype(jnp.bfloat16),
                   preferred_element_type=jnp.float32)


def _prep_body(h_ref, hs2_ref, t3_ref):
    h = h_ref[:, :HID]
    z = jnp.zeros_like(h)
    hs2_ref[...] = jnp.concatenate([z, h], axis=1)
    t3_ref[...] = jnp.concatenate([h, -h], axis=1)


def _prep(h128):
    n = h128.shape[0]
    bm = 1024
    return pl.pallas_call(
        _prep_body,
        grid=(n // bm,),
        in_specs=[pl.BlockSpec((bm, 128), lambda i: (i, 0))],
        out_specs=[
            pl.BlockSpec((bm, 128), lambda i: (i, 0)),
            pl.BlockSpec((bm, 128), lambda i: (i, 0)),
        ],
        out_shape=[
            jax.ShapeDtypeStruct((n, 128), jnp.float32),
            jax.ShapeDtypeStruct((n, 128), jnp.float32),
        ],
    )(h128)


def _mlp_body(m_ref, w1_ref, b1_ref, w2_ref, b2_ref, h_ref):
    p1 = _bfdot(m_ref[...], w1_ref[...]) + b1_ref[...]
    h_ref[...] = _bfdot(jax.nn.relu(p1), w2_ref[...]) + b2_ref[...]


def _edge_mlp(M, W1, b1, W2, b2):
    e, w = M.shape
    bm = 512
    return pl.pallas_call(
        _mlp_body,
        grid=(e // bm,),
        in_specs=[
            pl.BlockSpec((bm, w), lambda i: (i, 0)),
            pl.BlockSpec((w, HID), lambda i: (0, 0)),
            pl.BlockSpec((1, HID), lambda i: (0, 0)),
            pl.BlockSpec((HID, HID), lambda i: (0, 0)),
            pl.BlockSpec((1, HID), lambda i: (0, 0)),
        ],
        out_specs=pl.BlockSpec((bm, HID), lambda i: (i, 0)),
        out_shape=jax.ShapeDtypeStruct((EPAD, HID), jnp.float32),
    )(M, W1, b1[None, :], W2, b2[None, :])


def _mlp1_body(ml_ref, mr_ref, w1_ref, b1_ref, w2_ref, b2_ref, h_ref):
    m = jnp.concatenate([ml_ref[...], mr_ref[...]], axis=1)
    p1 = _bfdot(m, w1_ref[...]) + b1_ref[...]
    h_ref[...] = _bfdot(jax.nn.relu(p1), w2_ref[...]) + b2_ref[...]


def _edge_mlp1(ML, MR, W1, b1, W2, b2):
    e = ML.shape[0]
    bm = 512
    return pl.pallas_call(
        _mlp1_body,
        grid=(e // bm,),
        in_specs=[
            pl.BlockSpec((bm, 128), lambda i: (i, 0)),
            pl.BlockSpec((bm, 128), lambda i: (i, 0)),
            pl.BlockSpec((256, HID), lambda i: (0, 0)),
            pl.BlockSpec((1, HID), lambda i: (0, 0)),
            pl.BlockSpec((HID, HID), lambda i: (0, 0)),
            pl.BlockSpec((1, HID), lambda i: (0, 0)),
        ],
        out_specs=pl.BlockSpec((bm, HID), lambda i: (i, 0)),
        out_shape=jax.ShapeDtypeStruct((EPAD, HID), jnp.float32),
    )(ML, MR, W1, b1[None, :], W2, b2[None, :])


def _wid():
    return lax.axis_index("s") * 2 + lax.axis_index("c")


def _gather_body(hs2_hbm, t3_hbm, src_hbm, dst_hbm, m_hbm,
                 idxs0, idxd0, buf0, idxs1, idxd1, buf1, sem0, sem1):
    wid = _wid()
    ebase = wid * EPW

    def fire_plain(idxs_v, idxd_v, buf_v, sem, base):
        pltpu.sync_copy(src_hbm.at[pl.ds(base, GC)], idxs_v)
        pltpu.sync_copy(dst_hbm.at[pl.ds(base, GC)], idxd_v)
        hs = []
        for j in range(GC // GSUB):
            s = pl.ds(j * GSUB, GSUB)
            hs.append(
                pltpu.async_copy(hs2_hbm.at[idxs_v.at[s]], buf_v.at[s], sem))
        return hs

    def fire_add(idxd_v, buf_v, sem):
        hs = []
        for j in range(GC // GSUB):
            s = pl.ds(j * GSUB, GSUB)
            hs.append(
                pltpu.async_copy(t3_hbm.at[idxd_v.at[s]], buf_v.at[s], sem,
                                 add=True))
        return hs

    def pair(i, _):
        b0 = ebase + (2 * i) * GC
        b1 = b0 + GC
        p0 = fire_plain(idxs0, idxd0, buf0, sem0, b0)
        p1 = fire_plain(idxs1, idxd1, buf1, sem1, b1)
        for d in p0:
            d.wait()
        a0 = fire_add(idxd0, buf0, sem0)
        for d in p1:
            d.wait()
        a1 = fire_add(idxd1, buf1, sem1)
        for d in a0:
            d.wait()
        pltpu.sync_copy(buf0, m_hbm.at[pl.ds(b0, GC)])
        for d in a1:
            d.wait()
        pltpu.sync_copy(buf1, m_hbm.at[pl.ds(b1, GC)])
        return ()

    npair = EPW // (2 * GC)
    lax.fori_loop(0, npair, pair, (), unroll=False)
    base = ebase + 2 * npair * GC
    p0 = fire_plain(idxs0, idxd0, buf0, sem0, base)
    for d in p0:
        d.wait()
    a0 = fire_add(idxd0, buf0, sem0)
    for d in a0:
        d.wait()
    pltpu.sync_copy(buf0, m_hbm.at[pl.ds(base, GC)])


def _gather(HS2, T3, src_s, dst_s):
    mesh = plsc.VectorSubcoreMesh(core_axis_name="c", subcore_axis_name="s")
    kern = pl.kernel(
        _gather_body,
        out_type=jax.ShapeDtypeStruct((N_EDGES, 128), jnp.float32),
        mesh=mesh,
        scratch_types=[
            pltpu.VMEM((GC,), jnp.int32),
            pltpu.VMEM((GC,), jnp.int32),
            pltpu.VMEM((GC, 128), jnp.float32),
            pltpu.VMEM((GC,), jnp.int32),
            pltpu.VMEM((GC,), jnp.int32),
            pltpu.VMEM((GC, 128), jnp.float32),
            pltpu.SemaphoreType.DMA,
            pltpu.SemaphoreType.DMA,
        ],
    )
    return kern(HS2, T3, src_s, dst_s)


def _gather1_body(x_hbm, negx_hbm, src_hbm, dst_hbm, ml_hbm, mr_hbm,
                  idxs_v, idxd_v, bufl_v, bufr_v, sem1, sem2):
    wid = _wid()
    ebase = wid * EPW

    def chunk(i, _):
        base = ebase + i * GC
        pltpu.sync_copy(src_hbm.at[pl.ds(base, GC)], idxs_v)
        pltpu.sync_copy(dst_hbm.at[pl.ds(base, GC)], idxd_v)
        handles = []
        for j in range(GC // GSUB):
            s = pl.ds(j * GSUB, GSUB)
            handles.append(
                pltpu.async_copy(x_hbm.at[idxd_v.at[s]], bufl_v.at[s], sem1))
            handles.append(
                pltpu.async_copy(x_hbm.at[idxs_v.at[s]], bufr_v.at[s], sem2))
        for d in handles:
            d.wait()
        handles = []
        for j in range(GC // GSUB):
            s = pl.ds(j * GSUB, GSUB)
            handles.append(
                pltpu.async_copy(negx_hbm.at[idxd_v.at[s]], bufr_v.at[s], sem2,
                                 add=True))
        for d in handles:
            d.wait()
        pltpu.sync_copy(bufl_v, ml_hbm.at[pl.ds(base, GC)])
        pltpu.sync_copy(bufr_v, mr_hbm.at[pl.ds(base, GC)])
        return ()

    lax.fori_loop(0, EPW // GC, chunk, (), unroll=False)


def _gather1(xp, negx, src_s, dst_s):
    mesh = plsc.VectorSubcoreMesh(core_axis_name="c", subcore_axis_name="s")
    kern = pl.kernel(
        _gather1_body,
        out_type=[
            jax.ShapeDtypeStruct((N_EDGES, 128), jnp.float32),
            jax.ShapeDtypeStruct((N_EDGES, 128), jnp.float32),
        ],
        mesh=mesh,
        scratch_types=[
            pltpu.VMEM((GC,), jnp.int32),
            pltpu.VMEM((GC,), jnp.int32),
            pltpu.VMEM((GC, 128), jnp.float32),
            pltpu.VMEM((GC, 128), jnp.float32),
            pltpu.SemaphoreType.DMA,
            pltpu.SemaphoreType.DMA,
        ],
    )
    return kern(xp, negx, src_s, dst_s)


def _scatter_body(hm_hbm, dst_hbm, starts_hbm, out_hbm,
                  startv, idxb, hbuf, acc, sem):
    wid = _wid()
    pltpu.sync_copy(starts_hbm, startv)
    sv = startv[pl.ds(wid * 8, 16)]
    s0 = sv[0]
    s1 = sv[1]

    neg = jnp.full((16,), _NEG_INF, dtype=jnp.float32)

    def initrow(i, _):
        for c in range(8):
            acc[i, pl.ds(c * 16, 16)] = neg
        return ()

    lax.fori_loop(0, RPT + 1, initrow, (), unroll=False)

    eb0 = (s0 // 8) * 8
    nch = (s1 - eb0 + SC2 - 1) // SC2
    nbase = wid * RPT
    gi = lax.broadcasted_iota(jnp.int32, (16,), 0)

    def chunk(k, _):
        eb = eb0 + k * SC2
        pltpu.sync_copy(dst_hbm.at[pl.ds(eb, SC2)], idxb)
        pltpu.sync_copy(hm_hbm.at[pl.ds(eb, SC2)], hbuf)

        def group(g, _):
            idxv = idxb[pl.ds(g * 16, 16)] - nbase
            inb = (idxv >= 0) & (idxv < RPT) & ((eb + g * 16 + gi) < s1)
            rowv = jnp.where(inb, idxv, RPT)
            for l in range(16):
                row = rowv[l]
                for c in range(4):
                    hv = hbuf[g * 16 + l, pl.ds(c * 16, 16)]
                    av = acc[row, pl.ds(c * 16, 16)]
                    acc[row, pl.ds(c * 16, 16)] = jnp.maximum(av, hv)
            return ()

        lax.fori_loop(0, SC2 // 16, group, (), unroll=False)
        return ()

    lax.fori_loop(0, nch, chunk, (), unroll=False)

    zero = jnp.zeros((16,), jnp.float32)

    def fixrow(i, _):
        for c in range(8):
            v = acc[i, pl.ds(c * 16, 16)]
            acc[i, pl.ds(c * 16, 16)] = jnp.where(v == _NEG_INF, zero, v)
        return ()

    lax.fori_loop(0, RPT, fixrow, (), unroll=False)
    pltpu.sync_copy(acc.at[pl.ds(0, RPT)], out_hbm.at[pl.ds(wid * RPT, RPT)])


def _scatter_max(Hm, dst_s_pad, starts):
    mesh = plsc.VectorSubcoreMesh(core_axis_name="c", subcore_axis_name="s")
    kern = pl.kernel(
        _scatter_body,
        out_type=jax.ShapeDtypeStruct((NP, 128), jnp.float32),
        mesh=mesh,
        scratch_types=[
            pltpu.VMEM((272,), jnp.int32),
            pltpu.VMEM((SC2,), jnp.int32),
            pltpu.VMEM((SC2, HID), jnp.float32),
            pltpu.VMEM((RPT + 1, 128), jnp.float32),
            pltpu.SemaphoreType.DMA,
        ],
    )
    return kern(Hm, dst_s_pad, starts)


def _layer(h128, src_s, dst_s, dst_s_pad, starts, W1, b1, W2, b2):
    HS2, T3 = _prep(h128)
    M = _gather(HS2, T3, src_s, dst_s)
    Hm = _edge_mlp(M, W1, b1, W2, b2)
    return _scatter_max(Hm, dst_s_pad, starts)


def kernel(x, edge_index, edge_attr, We1, be1, We2, be2, Wc1, bc1, Wc2, bc2,
           Wd1, bd1, Wd2, bd2):
    src = edge_index[0].astype(jnp.int32)
    dst = edge_index[1].astype(jnp.int32)

    dst_s, order = lax.sort_key_val(dst, lax.iota(jnp.int32, N_EDGES))
    src_s = jnp.take(src, order)
    dst_s_pad = jnp.full((EPAD,), NP, jnp.int32).at[:N_EDGES].set(dst_s)
    bnds = jnp.searchsorted(dst_s, RPT * jnp.arange(33, dtype=jnp.int32),
                            side="left").astype(jnp.int32)
    t8 = 8 * jnp.arange(32, dtype=jnp.int32)
    starts = (jnp.zeros((272,), jnp.int32)
              .at[t8].set(bnds[:32])
              .at[t8 + 1].set(bnds[1:33]))

    xp = jnp.zeros((NP, 128), jnp.float32).at[:N_NODES].set(x)
    ML, MR = _gather1(xp, -xp, src_s, dst_s)
    Hm1 = _edge_mlp1(ML, MR, We1, be1, We2, be2)
    h = _scatter_max(Hm1, dst_s_pad, starts)

    for _ in range(8):
        h = _layer(h, src_s, dst_s, dst_s_pad, starts, Wc1, bc1, Wc2, bc2)

    Wd2p = jnp.zeros((HID, HID), jnp.float32).at[:, 0].set(Wd2[:, 0])
    bd2p = jnp.zeros((HID,), jnp.float32).at[0].set(bd2[0])
    h = _layer(h, src_s, dst_s, dst_s_pad, starts, Wd1, bd1, Wd2p, bd2p)

    out = h[:N_NODES, :1]
    return (out, edge_attr)
